# Initial kernel scaffold; baseline (speedup 1.0000x reference)
import jax
import jax.numpy as jnp
from jax.experimental import pallas as pl


def kernel(x, A_edge_vals, weight, gamma, beta, A_edge_index):
    raise NotImplementedError("write your pallas kernel here")


# speedup vs baseline: 3.8948x; 3.8948x over previous
import functools

import jax
import jax.numpy as jnp
from jax import lax
from jax.experimental import pallas as pl
from jax.experimental.pallas import tpu as pltpu
from jax.experimental.pallas import tpu_sc as plsc

N = 10000
E = 160000
D = 256
DH = 128
NC, NS, L = 2, 16, 16
CH = 80
EPT = E // NS
NCHUNK = EPT // CH
SLAB = NCHUNK
ROWS_PT = N // NS

_mesh = plsc.VectorSubcoreMesh(
    core_axis_name="c", subcore_axis_name="s", num_cores=NC, num_subcores=NS)


@functools.partial(
    pl.kernel,
    out_type=jax.ShapeDtypeStruct((NC, N, DH), jnp.float32),
    mesh=_mesh,
    compiler_params=pltpu.CompilerParams(use_tc_tiling_on_sc=False,
                                         needs_layout_passes=False),
    scratch_types=[
        pltpu.VMEM((SLAB, CH), jnp.int32),
        pltpu.VMEM((SLAB, CH), jnp.int32),
        pltpu.VMEM((SLAB, CH), jnp.float32),
        pltpu.VMEM((CH, DH), jnp.float32),
        pltpu.VMEM_SHARED((N, DH), jnp.float32),
        pltpu.SemaphoreType.DMA,
    ],
)
def _sc_aggregate(xparts, src2, dst2, vals2, zeros, out,
                  src_v, dst_v, vals_v, rows_v, acc, sem):
    c = lax.axis_index("c")
    s = lax.axis_index("s")

    pltpu.sync_copy(zeros, acc.at[pl.ds(s * ROWS_PT, ROWS_PT)])

    pltpu.sync_copy(src2.at[pl.ds(s * SLAB, SLAB)], src_v)
    pltpu.sync_copy(dst2.at[pl.ds(s * SLAB, SLAB)], dst_v)
    pltpu.sync_copy(vals2.at[pl.ds(s * SLAB, SLAB)], vals_v)


    off = jnp.full((L,), c * N, jnp.int32)

    def add_off(i, carry):
        for k in range(CH // L):
            src_v[i, pl.ds(k * L, L)] = src_v[i, pl.ds(k * L, L)] + off
        return carry

    lax.fori_loop(0, SLAB, add_off, 0)

    plsc.subcore_barrier()

    def chunk_body(j, carry):
        pltpu.async_copy(xparts.at[src_v.at[j]], rows_v, sem).wait()
        jrow = jnp.full((L,), j, jnp.int32)
        for e in range(CH):
            sp = plsc.load_gather(vals_v, [jrow, jnp.full((L,), e, jnp.int32)])
            for k in range(DH // L):
                sl = pl.ds(k * L, L)
                rows_v[e, sl] = rows_v[e, sl] * sp
        pltpu.sync_copy(rows_v, acc.at[dst_v.at[j]], add=True)
        return carry

    lax.fori_loop(0, NCHUNK, chunk_body, 0)

    plsc.subcore_barrier()

    pltpu.sync_copy(acc.at[pl.ds(s * ROWS_PT, ROWS_PT)],
                    out.at[c, pl.ds(s * ROWS_PT, ROWS_PT)])


BN = 1000


def _tc_body(sup_ref, x_ref, wa_ref, wb_ref, wc_ref, g_ref, b_ref, o_ref):
    acc = jnp.dot(sup_ref[0], wa_ref[...], preferred_element_type=jnp.float32)
    acc = acc + jnp.dot(sup_ref[1], wb_ref[...],
                        preferred_element_type=jnp.float32)
    acc = acc + jnp.dot(x_ref[...], wc_ref[...],
                        preferred_element_type=jnp.float32)
    mu = jnp.mean(acc, axis=-1, keepdims=True)
    d = acc - mu
    var = jnp.mean(d * d, axis=-1, keepdims=True)
    y = d * lax.rsqrt(var + 1e-5) * g_ref[...] + b_ref[...]
    o_ref[...] = jnp.maximum(y, 0.0)


_tc_dense = pl.pallas_call(
    _tc_body,
    grid=(N // BN,),
    in_specs=[
        pl.BlockSpec((NC, BN, DH), lambda i: (0, i, 0)),
        pl.BlockSpec((BN, D), lambda i: (i, 0)),
        pl.BlockSpec((DH, D), lambda i: (0, 0)),
        pl.BlockSpec((DH, D), lambda i: (0, 0)),
        pl.BlockSpec((D, D), lambda i: (0, 0)),
        pl.BlockSpec((1, D), lambda i: (0, 0)),
        pl.BlockSpec((1, D), lambda i: (0, 0)),
    ],
    out_specs=pl.BlockSpec((BN, D), lambda i: (i, 0)),
    out_shape=jax.ShapeDtypeStruct((N, D), jnp.float32),
)


def kernel(x, A_edge_vals, weight, gamma, beta, A_edge_index):
    src2 = A_edge_index[0].astype(jnp.int32).reshape(E // CH, CH)
    dst2 = A_edge_index[1].astype(jnp.int32).reshape(E // CH, CH)
    vals2 = A_edge_vals.reshape(E // CH, CH)
    xparts = jnp.concatenate([x[:, :DH], x[:, DH:]], axis=0)
    zeros = jnp.zeros((ROWS_PT, DH), jnp.float32)

    sup = _sc_aggregate(xparts, src2, dst2, vals2, zeros)

    wa = weight[:, :DH].T
    wb = weight[:, DH:2 * DH].T
    wc = weight[:, 2 * DH:].T
    return _tc_dense(sup, x, wa, wb, wc,
                     gamma.reshape(1, D), beta.reshape(1, D))
